# baseline (speedup 1.0000x reference)
import functools

import jax
import jax.numpy as jnp
from jax import lax
from jax.experimental import pallas as pl
from jax.experimental.pallas import tpu as pltpu
from jax.experimental.pallas import tpu_sc as plsc

HID = 64
IMG_H, IMG_W = 256, 1024
HW = IMG_H * IMG_W
N = 250000
NS = 80000
M = 80000
ZPAD = 8192
C = 512
NCH_PTS = (N + C - 1) // C
LAST_BASE = N - C
NCH_IMG = HW // C
NW = 32

G = 4
K = 8
JG = M // G
JCH = 4000
RS = 31264
NCOV = K * RS
RQ = HW // K


def _widx():
    return lax.axis_index("s") * 2 + lax.axis_index("c")


def _prep_loop(n_vregs, body, unroll=1):
    assert n_vregs % unroll == 0

    def step(i, c):
        for u in range(unroll):
            body(i * unroll + u)
        return c
    lax.fori_loop(0, n_vregs // unroll, step, 0)


def _win_scan(stag, wbuf, lo, span, jbase, n_vregs):
    iota = jnp.arange(16, dtype=jnp.int32)
    prev_sel = jnp.maximum(iota - 1, 0)
    dnums = lax.GatherDimensionNumbers(
        offset_dims=(), collapsed_slice_dims=(0,), start_index_map=(0,))

    def body(i):
        jv = jbase + i * 16 + iota
        idxv = stag[pl.ds(i * 16, 16)]
        m = (idxv >= lo) & (idxv < lo + span)
        comp = jnp.where(m, idxv * 16 + iota, -1)
        sk, sv = plsc.sort_key_val(comp, jv, descending=True)
        sidx = sk >> 4
        prev = lax.gather(sidx, prev_sel[:, None], dnums, (1,),
                          mode=lax.GatherScatterMode.PROMISE_IN_BOUNDS)
        keep = ((iota == 0) | (sidx != prev)) & (sk >= 0)
        addr = jnp.where(keep, sidx - lo, 0)
        plsc.store_scatter(wbuf, [addr], sv, mask=keep)
    _prep_loop(n_vregs, body, unroll=5)


def _win_body(idxs_hbm, idxp_hbm, pimg_hbm, ws_out, wp_out, wq_out, pix_out,
              stag, stag2, pixb, wbuf, sem_st):
    w = _widx()
    g = w // K
    mm = w % K
    iota = jnp.arange(16, dtype=jnp.int32)
    neg1 = jnp.full((16,), -1, jnp.int32)
    nch = JG // JCH

    def fill(i):
        wbuf[pl.ds(i * 16, 16)] = neg1

    def scan_array(idx_hbm, span):
        d = pltpu.make_async_copy(
            idx_hbm.at[pl.ds(g * JG, JCH)], stag.at[0], sem_st)
        d.start()
        for ch in range(nch):
            b = ch % 2
            d.wait()
            if ch + 1 < nch:
                d = pltpu.make_async_copy(
                    idx_hbm.at[pl.ds(g * JG + (ch + 1) * JCH, JCH)],
                    stag.at[(ch + 1) % 2], sem_st)
                d.start()
            _win_scan(stag.at[b], wbuf, mm * span, span,
                      g * JG + ch * JCH, JCH // 16)

    _prep_loop(RS // 16, fill, unroll=2)
    scan_array(idxs_hbm, RS)
    pltpu.sync_copy(wbuf.at[pl.ds(0, RS)], ws_out.at[g, pl.ds(mm * RS, RS)])

    _prep_loop(RS // 16, fill, unroll=2)
    scan_array(idxp_hbm, RS)
    pltpu.sync_copy(wbuf.at[pl.ds(0, RS)], wp_out.at[g, pl.ds(mm * RS, RS)])

    _prep_loop(RQ // 16, fill, unroll=2)
    d = pltpu.make_async_copy(
        pimg_hbm.at[pl.ds(g * JG, JCH), :], stag2.at[0], sem_st)
    d.start()
    for ch in range(nch):
        b = ch % 2
        j0 = g * JG + ch * JCH
        d.wait()
        if ch + 1 < nch:
            d = pltpu.make_async_copy(
                pimg_hbm.at[pl.ds(j0 + JCH, JCH), :], stag2.at[(ch + 1) % 2],
                sem_st)
            d.start()

        def mkpix(i):
            ridx = i * 16 + iota
            zero = jnp.zeros((16,), jnp.int32)
            rv = plsc.load_gather(stag2.at[b], [ridx, zero])
            cv = plsc.load_gather(stag2.at[b], [ridx, zero + 1])
            pixb[pl.ds(i * 16, 16)] = rv * IMG_W + cv
        _prep_loop(JCH // 16, mkpix, unroll=5)

        @pl.when(mm == 0)
        def _():
            pltpu.sync_copy(pixb, pix_out.at[pl.ds(j0, JCH)])
        _win_scan(pixb, wbuf, mm * RQ, RQ, j0, JCH // 16)
    pltpu.sync_copy(wbuf, wq_out.at[g, pl.ds(mm * RQ, RQ)])


def _max4(buf, sl):
    return jnp.maximum(jnp.maximum(buf[0, sl], buf[1, sl]),
                       jnp.maximum(buf[2, sl], buf[3, sl]))


NT_PTS = (NCH_PTS + NW - 1) // NW


def _pts_body(wsp_hbm, wpp_hbm, pix_hbm, pts_ext, imgt_ext,
              out_hbm, cls_hbm, wsf_hbm,
              wst_s, wst_p, wsv, wpv, idx1, pixv, sidx, iidx,
              bufa, bufb, clsb, wsw,
              sem_ws, sem_wp, sem_pix, sem_ga, sem_gb, sem_wr):
    w = _widx()
    iota = jnp.arange(16, dtype=jnp.int32)

    def chunk_info(t):
        kk = w + NW * t
        base = jnp.where(kk == NCH_PTS - 1, LAST_BASE, C * kk)
        return kk < NCH_PTS, base

    def mk_wloads(t):
        valid, base = chunk_info(t)
        dls = pltpu.make_async_copy(wsp_hbm.at[:, pl.ds(base, C)], wst_s, sem_ws)
        dlp = pltpu.make_async_copy(wpp_hbm.at[:, pl.ds(base, C)], wst_p, sem_wp)
        return valid, base, dls, dlp

    def stage_a(t):
        valid, base, dls, dlp = mk_wloads(t)
        b = t % 2

        @pl.when(valid)
        def _():
            dls.wait()
            dlp.wait()

            def comb(i):
                sl = pl.ds(i * 16, 16)
                wsv[b, sl] = _max4(wst_s, sl)
                wpv[b, sl] = _max4(wst_p, sl)
            _prep_loop(C // 16, comb)

            def prep1(i):
                rid = base + i * 16 + iota
                wp = wpv[b, pl.ds(i * 16, 16)]
                idx1[b, pl.ds(i * 16, 16)] = jnp.where(wp >= 0, wp,
                                                       rid & 8191)
            _prep_loop(C // 16, prep1)
        d = pltpu.make_async_copy(pix_hbm.at[idx1.at[b]], pixv.at[b],
                                  sem_pix)

        @pl.when(valid)
        def _():
            d.start()
        nxt = mk_wloads(t + 1) if t + 1 < NT_PTS else None
        if nxt is not None:

            @pl.when(nxt[0])
            def _():
                nxt[2].start()
                nxt[3].start()
        return d

    first = mk_wloads(0)

    @pl.when(first[0])
    def _():
        first[2].start()
        first[3].start()

    d_pix = stage_a(0)
    wr_descs = [None] * NT_PTS
    for t in range(NT_PTS):
        valid, base = chunk_info(t)
        b = t % 2

        if t >= 2:
            pvalid, descs = wr_descs[t - 2]

            @pl.when(pvalid)
            def _(descs=descs):
                for d in descs:
                    d.wait()

        d_pix_cur = d_pix
        if t + 1 < NT_PTS:
            d_pix = stage_a(t + 1)

        @pl.when(valid)
        def _(valid=valid, base=base, b=b):
            d_pix_cur.wait()

            def prep2(i):
                rid = base + i * 16 + iota
                ws = wsv[b, pl.ds(i * 16, 16)]
                wp = wpv[b, pl.ds(i * 16, 16)]
                pv = pixv[b, pl.ds(i * 16, 16)]
                sidx[pl.ds(i * 16, 16)] = jnp.where(
                    ws >= 0, ws, NS + (rid & (ZPAD - 1)))
                iidx[pl.ds(i * 16, 16)] = jnp.where(
                    wp >= 0, pv, HW + (rid & (ZPAD - 1)))
                clsb[b, pl.ds(i * 16, 16)] = jnp.where(
                    ws >= 0, jnp.float32(0.0), jnp.float32(1.0))
                wsw[b, pl.ds(i * 16, 16)] = ws
            _prep_loop(C // 16, prep2)

        d_ga = pltpu.make_async_copy(pts_ext.at[sidx], bufa.at[b], sem_ga)
        d_gb = pltpu.make_async_copy(imgt_ext.at[iidx], bufb, sem_gb)

        @pl.when(valid)
        def _(b=b):
            d_ga.start()
            d_gb.start()
            d_ga.wait()
            d_gb.wait()

            def add_row(r):
                for c in range(HID // 16):
                    sl = pl.ds(c * 16, 16)
                    bufa[b, r, sl] = bufa[b, r, sl] + bufb[r, sl]
            _prep_loop(C, add_row)

        dw1 = pltpu.make_async_copy(bufa.at[b], out_hbm.at[pl.ds(base, C)],
                                    sem_wr)
        dw2 = pltpu.make_async_copy(clsb.at[b], cls_hbm.at[pl.ds(base, C)],
                                    sem_wr)
        dw3 = pltpu.make_async_copy(wsw.at[b], wsf_hbm.at[pl.ds(base, C)],
                                    sem_wr)

        @pl.when(valid)
        def _():
            dw1.start()
            dw2.start()
            dw3.start()
        wr_descs[t] = (valid, (dw1, dw2, dw3))

    for t in (NT_PTS - 2, NT_PTS - 1):
        pvalid, descs = wr_descs[t]

        @pl.when(pvalid)
        def _(descs=descs):
            for d in descs:
                d.wait()


NT_IMG = NCH_IMG // NW


def _img_body(wqp_hbm, idxp_hbm, ws_hbm, pts_ext, out_hbm,
              wst, wqv, idx1, pvv, ws2v, idx3, bufa,
              sem_w, sem_pv, sem_ws2, sem_ga, sem_wr):
    w = _widx()
    iota = jnp.arange(16, dtype=jnp.int32)

    def base_of(t):
        return C * (w + NW * t)

    def mk_wload(t):
        return pltpu.make_async_copy(
            wqp_hbm.at[:, pl.ds(base_of(t), C)], wst, sem_w)

    def stage_a(t, d_wl):
        b = t % 2
        d_wl.wait()
        base = base_of(t)

        def comb(i):
            sl = pl.ds(i * 16, 16)
            wqv[b, sl] = _max4(wst, sl)
        _prep_loop(C // 16, comb)

        def prep1(i):
            rid = base + i * 16 + iota
            wq = wqv[b, pl.ds(i * 16, 16)]
            idx1[b, pl.ds(i * 16, 16)] = jnp.where(wq >= 0, wq, rid & 8191)
        _prep_loop(C // 16, prep1)
        d = pltpu.make_async_copy(idxp_hbm.at[idx1.at[b]], pvv.at[b], sem_pv)
        d.start()
        return d

    d_wl = mk_wload(0)
    d_wl.start()
    d_pv = stage_a(0, d_wl)
    d_wl = mk_wload(1)
    d_wl.start()

    wr_descs = [None] * NT_IMG
    for t in range(NT_IMG):
        b = t % 2
        base = base_of(t)

        d_pv_cur = d_pv
        if t + 1 < NT_IMG:
            d_pv = stage_a(t + 1, d_wl)
            if t + 2 < NT_IMG:
                d_wl = mk_wload(t + 2)
                d_wl.start()

        d_pv_cur.wait()

        def prep2(i):
            rid = base + i * 16 + iota
            wq = wqv[b, pl.ds(i * 16, 16)]
            p = pvv[b, pl.ds(i * 16, 16)]
            idx1[b, pl.ds(i * 16, 16)] = jnp.where(wq >= 0, p, rid & 131071)
        _prep_loop(C // 16, prep2)
        d_ws2 = pltpu.make_async_copy(ws_hbm.at[idx1.at[b]], ws2v, sem_ws2)
        d_ws2.start()
        d_ws2.wait()

        def prep3(i):
            rid = base + i * 16 + iota
            wq = wqv[b, pl.ds(i * 16, 16)]
            ws = ws2v[pl.ds(i * 16, 16)]
            live = (wq >= 0) & (ws >= 0)
            idx3[pl.ds(i * 16, 16)] = jnp.where(
                live, ws, NS + (rid & (ZPAD - 1)))
        _prep_loop(C // 16, prep3)

        if t >= 2:
            wr_descs[t - 2].wait()
        d_ga = pltpu.make_async_copy(pts_ext.at[idx3], bufa.at[b], sem_ga)
        d_ga.start()
        d_ga.wait()
        d_wr = pltpu.make_async_copy(bufa.at[b], out_hbm.at[pl.ds(base, C)],
                                     sem_wr)
        d_wr.start()
        wr_descs[t] = d_wr

    wr_descs[NT_IMG - 2].wait()
    wr_descs[NT_IMG - 1].wait()


def _tr_body(src_ref, out_ref):
    i = pl.program_id(0)

    @pl.when(i < HW // 8192)
    def _():
        x = src_ref[...].reshape(HID, 8192)
        out_ref[...] = jnp.transpose(x, (1, 0))

    @pl.when(i >= HW // 8192)
    def _():
        out_ref[...] = jnp.zeros((8192, HID), jnp.float32)


_tr_call = pl.pallas_call(
    _tr_body,
    grid=((HW + ZPAD) // 8192,),
    in_specs=[pl.BlockSpec((HID, 8, IMG_W),
                           lambda i: (0, jnp.minimum(i, HW // 8192 - 1), 0))],
    out_specs=pl.BlockSpec((8192, HID), lambda i: (i, 0)),
    out_shape=jax.ShapeDtypeStruct((HW + ZPAD, HID), jnp.float32),
)

_MESH = plsc.VectorSubcoreMesh(core_axis_name="c", subcore_axis_name="s")
_PARAMS = pltpu.CompilerParams(use_tc_tiling_on_sc=False,
                               needs_layout_passes=False)

_win_call = functools.partial(
    pl.kernel,
    compiler_params=_PARAMS,
    out_type=(jax.ShapeDtypeStruct((G, NCOV), jnp.int32),
              jax.ShapeDtypeStruct((G, NCOV), jnp.int32),
              jax.ShapeDtypeStruct((G, HW), jnp.int32),
              jax.ShapeDtypeStruct((M,), jnp.int32)),
    mesh=_MESH,
    scratch_types=[
        pltpu.VMEM((2, JCH), jnp.int32),
        pltpu.VMEM((2, JCH, 2), jnp.int32),
        pltpu.VMEM((JCH,), jnp.int32),
        pltpu.VMEM((RQ,), jnp.int32),
        pltpu.SemaphoreType.DMA,
    ],
)(_win_body)

_pts_call = functools.partial(
    pl.kernel,
    compiler_params=_PARAMS,
    out_type=(jax.ShapeDtypeStruct((N, HID), jnp.float32),
              jax.ShapeDtypeStruct((N,), jnp.float32),
              jax.ShapeDtypeStruct((N,), jnp.int32)),
    mesh=_MESH,
    scratch_types=[
        pltpu.VMEM((G, C), jnp.int32),
        pltpu.VMEM((G, C), jnp.int32),
        pltpu.VMEM((2, C), jnp.int32),
        pltpu.VMEM((2, C), jnp.int32),
        pltpu.VMEM((2, C), jnp.int32),
        pltpu.VMEM((2, C), jnp.int32),
        pltpu.VMEM((C,), jnp.int32),
        pltpu.VMEM((C,), jnp.int32),
        pltpu.VMEM((2, C, HID), jnp.float32),
        pltpu.VMEM((C, HID), jnp.float32),
        pltpu.VMEM((2, C), jnp.float32),
        pltpu.VMEM((2, C), jnp.int32),
        pltpu.SemaphoreType.DMA,
        pltpu.SemaphoreType.DMA,
        pltpu.SemaphoreType.DMA,
        pltpu.SemaphoreType.DMA,
        pltpu.SemaphoreType.DMA,
        pltpu.SemaphoreType.DMA,
    ],
)(_pts_body)

_img_call = functools.partial(
    pl.kernel,
    compiler_params=_PARAMS,
    out_type=jax.ShapeDtypeStruct((HW, HID), jnp.float32),
    mesh=_MESH,
    scratch_types=[
        pltpu.VMEM((G, C), jnp.int32),
        pltpu.VMEM((2, C), jnp.int32),
        pltpu.VMEM((2, C), jnp.int32),
        pltpu.VMEM((2, C), jnp.int32),
        pltpu.VMEM((C,), jnp.int32),
        pltpu.VMEM((C,), jnp.int32),
        pltpu.VMEM((2, C, HID), jnp.float32),
        pltpu.SemaphoreType.DMA,
        pltpu.SemaphoreType.DMA,
        pltpu.SemaphoreType.DMA,
        pltpu.SemaphoreType.DMA,
        pltpu.SemaphoreType.DMA,
    ],
)(_img_body)


def kernel(pts_fea, p2img_idx, batch_idx, pts_fea_sample, batch_idx_sample,
           sample_index, img_latent_full, points_img):
    idx_s = sample_index.reshape(NS)
    idx_p = p2img_idx.reshape(M)
    pimg = points_img.reshape(M, 2)

    pts_ext = jnp.pad(pts_fea_sample, ((0, ZPAD), (0, 0)))
    imgt_ext = _tr_call(img_latent_full[0])

    wsp, wpp, wqp, pix = _win_call(idx_s, idx_p, pimg)
    out_pts, cls, ws_final = _pts_call(wsp, wpp, pix, pts_ext, imgt_ext)
    img_flat = _img_call(wqp, idx_p, ws_final, pts_ext)

    return (out_pts, cls.reshape(N, 1),
            img_flat.reshape(1, IMG_H, IMG_W, HID))


# speedup vs baseline: 1.0079x; 1.0079x over previous
import functools

import jax
import jax.numpy as jnp
from jax import lax
from jax.experimental import pallas as pl
from jax.experimental.pallas import tpu as pltpu
from jax.experimental.pallas import tpu_sc as plsc

HID = 64
IMG_H, IMG_W = 256, 1024
HW = IMG_H * IMG_W
N = 250000
NS = 80000
M = 80000
ZPAD = 8192
C = 512
NCH_PTS = (N + C - 1) // C
LAST_BASE = N - C
NCH_IMG = HW // C
NW = 32

G = 4
K = 8
JG = M // G
JCH = 4000
RS = 31264
NCOV = K * RS
RQ = HW // K


def _widx():
    return lax.axis_index("s") * 2 + lax.axis_index("c")


def _prep_loop(n_vregs, body, unroll=1):
    assert n_vregs % unroll == 0

    def step(i, c):
        for u in range(unroll):
            body(i * unroll + u)
        return c
    lax.fori_loop(0, n_vregs // unroll, step, 0)


def _win_scan(stag, wbuf, lo, span, jbase, n_vregs):
    iota = jnp.arange(16, dtype=jnp.int32)
    prev_sel = jnp.maximum(iota - 1, 0)
    dnums = lax.GatherDimensionNumbers(
        offset_dims=(), collapsed_slice_dims=(0,), start_index_map=(0,))

    def body(i):
        jv = jbase + i * 16 + iota
        idxv = stag[pl.ds(i * 16, 16)]
        m = (idxv >= lo) & (idxv < lo + span)
        comp = jnp.where(m, idxv * 16 + iota, -1)
        sk, sv = plsc.sort_key_val(comp, jv, descending=True)
        sidx = sk >> 4
        prev = lax.gather(sidx, prev_sel[:, None], dnums, (1,),
                          mode=lax.GatherScatterMode.PROMISE_IN_BOUNDS)
        keep = ((iota == 0) | (sidx != prev)) & (sk >= 0)
        addr = jnp.where(keep, sidx - lo, 0)
        plsc.store_scatter(wbuf, [addr], sv, mask=keep)
    _prep_loop(n_vregs, body, unroll=5)


def _win_body(idxs_hbm, idxp_hbm, pimg_hbm, ws_out, wp_out, wq_out, pix_out,
              stag, stag2, pixb, wbuf, sem_st):
    w = _widx()
    g = w // K
    mm = w % K
    iota = jnp.arange(16, dtype=jnp.int32)
    neg1 = jnp.full((16,), -1, jnp.int32)
    nch = JG // JCH

    def fill(i):
        wbuf[pl.ds(i * 16, 16)] = neg1

    def scan_array(idx_hbm, span):
        for ch in range(nch):
            j0 = g * JG + ch * JCH
            pltpu.sync_copy(idx_hbm.at[pl.ds(j0, JCH)], stag.at[0])
            _win_scan(stag.at[0], wbuf, mm * span, span, j0, JCH // 16)

    _prep_loop(RS // 16, fill, unroll=2)
    scan_array(idxs_hbm, RS)
    pltpu.sync_copy(wbuf.at[pl.ds(0, RS)], ws_out.at[g, pl.ds(mm * RS, RS)])

    _prep_loop(RS // 16, fill, unroll=2)
    scan_array(idxp_hbm, RS)
    pltpu.sync_copy(wbuf.at[pl.ds(0, RS)], wp_out.at[g, pl.ds(mm * RS, RS)])

    _prep_loop(RQ // 16, fill, unroll=2)
    for ch in range(nch):
        j0 = g * JG + ch * JCH
        pltpu.sync_copy(pimg_hbm.at[pl.ds(j0, JCH), :], stag2.at[0])

        def mkpix(i):
            ridx = i * 16 + iota
            zero = jnp.zeros((16,), jnp.int32)
            rv = plsc.load_gather(stag2.at[0], [ridx, zero])
            cv = plsc.load_gather(stag2.at[0], [ridx, zero + 1])
            pixb[pl.ds(i * 16, 16)] = rv * IMG_W + cv
        _prep_loop(JCH // 16, mkpix, unroll=5)

        @pl.when(mm == 0)
        def _():
            pltpu.sync_copy(pixb, pix_out.at[pl.ds(j0, JCH)])
        _win_scan(pixb, wbuf, mm * RQ, RQ, j0, JCH // 16)
    pltpu.sync_copy(wbuf, wq_out.at[g, pl.ds(mm * RQ, RQ)])


def _max4(buf, sl):
    return jnp.maximum(jnp.maximum(buf[0, sl], buf[1, sl]),
                       jnp.maximum(buf[2, sl], buf[3, sl]))


NT_PTS = (NCH_PTS + NW - 1) // NW


def _pts_body(wsp_hbm, wpp_hbm, pix_hbm, pts_ext, imgt_ext,
              out_hbm, cls_hbm, wsf_hbm,
              wst_s, wst_p, wsv, wpv, idx1, pixv, sidx, iidx,
              bufa, bufb, clsb, wsw,
              sem_ws, sem_wp, sem_pix, sem_ga, sem_gb, sem_wr):
    w = _widx()
    iota = jnp.arange(16, dtype=jnp.int32)

    def chunk_info(t):
        kk = w + NW * t
        base = jnp.where(kk == NCH_PTS - 1, LAST_BASE, C * kk)
        return kk < NCH_PTS, base

    def mk_wloads(t):
        valid, base = chunk_info(t)
        dls = pltpu.make_async_copy(wsp_hbm.at[:, pl.ds(base, C)], wst_s, sem_ws)
        dlp = pltpu.make_async_copy(wpp_hbm.at[:, pl.ds(base, C)], wst_p, sem_wp)
        return valid, base, dls, dlp

    def stage_a(t):
        valid, base, dls, dlp = mk_wloads(t)
        b = t % 2

        @pl.when(valid)
        def _():
            dls.wait()
            dlp.wait()

            def comb(i):
                sl = pl.ds(i * 16, 16)
                wsv[b, sl] = _max4(wst_s, sl)
                wpv[b, sl] = _max4(wst_p, sl)
            _prep_loop(C // 16, comb)

            def prep1(i):
                rid = base + i * 16 + iota
                wp = wpv[b, pl.ds(i * 16, 16)]
                idx1[b, pl.ds(i * 16, 16)] = jnp.where(wp >= 0, wp,
                                                       rid & 8191)
            _prep_loop(C // 16, prep1)
        d = pltpu.make_async_copy(pix_hbm.at[idx1.at[b]], pixv.at[b],
                                  sem_pix)

        @pl.when(valid)
        def _():
            d.start()
        nxt = mk_wloads(t + 1) if t + 1 < NT_PTS else None
        if nxt is not None:

            @pl.when(nxt[0])
            def _():
                nxt[2].start()
                nxt[3].start()
        return d

    first = mk_wloads(0)

    @pl.when(first[0])
    def _():
        first[2].start()
        first[3].start()

    d_pix = stage_a(0)
    wr_descs = [None] * NT_PTS
    for t in range(NT_PTS):
        valid, base = chunk_info(t)
        b = t % 2

        if t >= 2:
            pvalid, descs = wr_descs[t - 2]

            @pl.when(pvalid)
            def _(descs=descs):
                for d in descs:
                    d.wait()

        d_pix_cur = d_pix
        if t + 1 < NT_PTS:
            d_pix = stage_a(t + 1)

        @pl.when(valid)
        def _(valid=valid, base=base, b=b):
            d_pix_cur.wait()

            def prep2(i):
                rid = base + i * 16 + iota
                ws = wsv[b, pl.ds(i * 16, 16)]
                wp = wpv[b, pl.ds(i * 16, 16)]
                pv = pixv[b, pl.ds(i * 16, 16)]
                sidx[pl.ds(i * 16, 16)] = jnp.where(
                    ws >= 0, ws, NS + (rid & (ZPAD - 1)))
                iidx[pl.ds(i * 16, 16)] = jnp.where(
                    wp >= 0, pv, HW + (rid & (ZPAD - 1)))
                clsb[b, pl.ds(i * 16, 16)] = jnp.where(
                    ws >= 0, jnp.float32(0.0), jnp.float32(1.0))
                wsw[b, pl.ds(i * 16, 16)] = ws
            _prep_loop(C // 16, prep2)

        d_ga = pltpu.make_async_copy(pts_ext.at[sidx], bufa.at[b], sem_ga)
        d_gb = pltpu.make_async_copy(imgt_ext.at[iidx], bufb, sem_gb)

        @pl.when(valid)
        def _(b=b):
            d_ga.start()
            d_gb.start()
            d_ga.wait()
            d_gb.wait()

            def add_row(r):
                for c in range(HID // 16):
                    sl = pl.ds(c * 16, 16)
                    bufa[b, r, sl] = bufa[b, r, sl] + bufb[r, sl]
            _prep_loop(C, add_row)

        dw1 = pltpu.make_async_copy(bufa.at[b], out_hbm.at[pl.ds(base, C)],
                                    sem_wr)
        dw2 = pltpu.make_async_copy(clsb.at[b], cls_hbm.at[pl.ds(base, C)],
                                    sem_wr)
        dw3 = pltpu.make_async_copy(wsw.at[b], wsf_hbm.at[pl.ds(base, C)],
                                    sem_wr)

        @pl.when(valid)
        def _():
            dw1.start()
            dw2.start()
            dw3.start()
        wr_descs[t] = (valid, (dw1, dw2, dw3))

    for t in (NT_PTS - 2, NT_PTS - 1):
        pvalid, descs = wr_descs[t]

        @pl.when(pvalid)
        def _(descs=descs):
            for d in descs:
                d.wait()


NT_IMG = NCH_IMG // NW


def _img_body(wqp_hbm, idxp_hbm, ws_hbm, pts_ext, out_hbm,
              wst, wqv, idx1, pvv, ws2v, idx3, bufa,
              sem_w, sem_pv, sem_ws2, sem_ga, sem_wr):
    w = _widx()
    iota = jnp.arange(16, dtype=jnp.int32)

    def base_of(t):
        return C * (w + NW * t)

    def mk_wload(t):
        return pltpu.make_async_copy(
            wqp_hbm.at[:, pl.ds(base_of(t), C)], wst, sem_w)

    def stage_a(t, d_wl):
        b = t % 2
        d_wl.wait()
        base = base_of(t)

        def comb(i):
            sl = pl.ds(i * 16, 16)
            wqv[b, sl] = _max4(wst, sl)
        _prep_loop(C // 16, comb)

        def prep1(i):
            rid = base + i * 16 + iota
            wq = wqv[b, pl.ds(i * 16, 16)]
            idx1[b, pl.ds(i * 16, 16)] = jnp.where(wq >= 0, wq, rid & 8191)
        _prep_loop(C // 16, prep1)
        d = pltpu.make_async_copy(idxp_hbm.at[idx1.at[b]], pvv.at[b], sem_pv)
        d.start()
        return d

    d_wl = mk_wload(0)
    d_wl.start()
    d_pv = stage_a(0, d_wl)
    d_wl = mk_wload(1)
    d_wl.start()

    wr_descs = [None] * NT_IMG
    for t in range(NT_IMG):
        b = t % 2
        base = base_of(t)

        d_pv_cur = d_pv
        if t + 1 < NT_IMG:
            d_pv = stage_a(t + 1, d_wl)
            if t + 2 < NT_IMG:
                d_wl = mk_wload(t + 2)
                d_wl.start()

        d_pv_cur.wait()

        def prep2(i):
            rid = base + i * 16 + iota
            wq = wqv[b, pl.ds(i * 16, 16)]
            p = pvv[b, pl.ds(i * 16, 16)]
            idx1[b, pl.ds(i * 16, 16)] = jnp.where(wq >= 0, p, rid & 131071)
        _prep_loop(C // 16, prep2)
        d_ws2 = pltpu.make_async_copy(ws_hbm.at[idx1.at[b]], ws2v, sem_ws2)
        d_ws2.start()
        d_ws2.wait()

        def prep3(i):
            rid = base + i * 16 + iota
            wq = wqv[b, pl.ds(i * 16, 16)]
            ws = ws2v[pl.ds(i * 16, 16)]
            live = (wq >= 0) & (ws >= 0)
            idx3[pl.ds(i * 16, 16)] = jnp.where(
                live, ws, NS + (rid & (ZPAD - 1)))
        _prep_loop(C // 16, prep3)

        if t >= 2:
            wr_descs[t - 2].wait()
        d_ga = pltpu.make_async_copy(pts_ext.at[idx3], bufa.at[b], sem_ga)
        d_ga.start()
        d_ga.wait()
        d_wr = pltpu.make_async_copy(bufa.at[b], out_hbm.at[pl.ds(base, C)],
                                     sem_wr)
        d_wr.start()
        wr_descs[t] = d_wr

    wr_descs[NT_IMG - 2].wait()
    wr_descs[NT_IMG - 1].wait()


def _tr_body(src_ref, out_ref):
    i = pl.program_id(0)

    @pl.when(i < HW // 8192)
    def _():
        x = src_ref[...].reshape(HID, 8192)
        out_ref[...] = jnp.transpose(x, (1, 0))

    @pl.when(i >= HW // 8192)
    def _():
        out_ref[...] = jnp.zeros((8192, HID), jnp.float32)


_tr_call = pl.pallas_call(
    _tr_body,
    grid=((HW + ZPAD) // 8192,),
    in_specs=[pl.BlockSpec((HID, 8, IMG_W),
                           lambda i: (0, jnp.minimum(i, HW // 8192 - 1), 0))],
    out_specs=pl.BlockSpec((8192, HID), lambda i: (i, 0)),
    out_shape=jax.ShapeDtypeStruct((HW + ZPAD, HID), jnp.float32),
)

_MESH = plsc.VectorSubcoreMesh(core_axis_name="c", subcore_axis_name="s")
_PARAMS = pltpu.CompilerParams(use_tc_tiling_on_sc=False,
                               needs_layout_passes=False)

_win_call = functools.partial(
    pl.kernel,
    compiler_params=_PARAMS,
    out_type=(jax.ShapeDtypeStruct((G, NCOV), jnp.int32),
              jax.ShapeDtypeStruct((G, NCOV), jnp.int32),
              jax.ShapeDtypeStruct((G, HW), jnp.int32),
              jax.ShapeDtypeStruct((M,), jnp.int32)),
    mesh=_MESH,
    scratch_types=[
        pltpu.VMEM((2, JCH), jnp.int32),
        pltpu.VMEM((2, JCH, 2), jnp.int32),
        pltpu.VMEM((JCH,), jnp.int32),
        pltpu.VMEM((RQ,), jnp.int32),
        pltpu.SemaphoreType.DMA,
    ],
)(_win_body)

_pts_call = functools.partial(
    pl.kernel,
    compiler_params=_PARAMS,
    out_type=(jax.ShapeDtypeStruct((N, HID), jnp.float32),
              jax.ShapeDtypeStruct((N,), jnp.float32),
              jax.ShapeDtypeStruct((N,), jnp.int32)),
    mesh=_MESH,
    scratch_types=[
        pltpu.VMEM((G, C), jnp.int32),
        pltpu.VMEM((G, C), jnp.int32),
        pltpu.VMEM((2, C), jnp.int32),
        pltpu.VMEM((2, C), jnp.int32),
        pltpu.VMEM((2, C), jnp.int32),
        pltpu.VMEM((2, C), jnp.int32),
        pltpu.VMEM((C,), jnp.int32),
        pltpu.VMEM((C,), jnp.int32),
        pltpu.VMEM((2, C, HID), jnp.float32),
        pltpu.VMEM((C, HID), jnp.float32),
        pltpu.VMEM((2, C), jnp.float32),
        pltpu.VMEM((2, C), jnp.int32),
        pltpu.SemaphoreType.DMA,
        pltpu.SemaphoreType.DMA,
        pltpu.SemaphoreType.DMA,
        pltpu.SemaphoreType.DMA,
        pltpu.SemaphoreType.DMA,
        pltpu.SemaphoreType.DMA,
    ],
)(_pts_body)

_img_call = functools.partial(
    pl.kernel,
    compiler_params=_PARAMS,
    out_type=jax.ShapeDtypeStruct((HW, HID), jnp.float32),
    mesh=_MESH,
    scratch_types=[
        pltpu.VMEM((G, C), jnp.int32),
        pltpu.VMEM((2, C), jnp.int32),
        pltpu.VMEM((2, C), jnp.int32),
        pltpu.VMEM((2, C), jnp.int32),
        pltpu.VMEM((C,), jnp.int32),
        pltpu.VMEM((C,), jnp.int32),
        pltpu.VMEM((2, C, HID), jnp.float32),
        pltpu.SemaphoreType.DMA,
        pltpu.SemaphoreType.DMA,
        pltpu.SemaphoreType.DMA,
        pltpu.SemaphoreType.DMA,
        pltpu.SemaphoreType.DMA,
    ],
)(_img_body)


def kernel(pts_fea, p2img_idx, batch_idx, pts_fea_sample, batch_idx_sample,
           sample_index, img_latent_full, points_img):
    idx_s = sample_index.reshape(NS)
    idx_p = p2img_idx.reshape(M)
    pimg = points_img.reshape(M, 2)

    pts_ext = jnp.pad(pts_fea_sample, ((0, ZPAD), (0, 0)))
    imgt_ext = _tr_call(img_latent_full[0])

    wsp, wpp, wqp, pix = _win_call(idx_s, idx_p, pimg)
    out_pts, cls, ws_final = _pts_call(wsp, wpp, pix, pts_ext, imgt_ext)
    img_flat = _img_call(wqp, idx_p, ws_final, pts_ext)

    return (out_pts, cls.reshape(N, 1),
            img_flat.reshape(1, IMG_H, IMG_W, HID))
